# baseline (speedup 1.0000x reference)
import jax
import jax.numpy as jnp
from jax.experimental import pallas as pl
from jax.experimental.pallas import tpu as pltpu

_BM = 2048
_CHUNK = 1024


def _dot_t(w, a):
    return jax.lax.dot_general(
        w, a, dimension_numbers=(((0,), (0,)), ((), ())),
        precision=jax.lax.Precision.DEFAULT,
        preferred_element_type=jnp.float32)


def _encoder_body(x_ref, w1_ref, b1_ref, w2_ref, b2_ref, o_ref):
    w1 = w1_ref[...]
    b1 = b1_ref[...]
    w2 = w2_ref[...]
    b2 = b2_ref[...]
    for j in range(_BM // _CHUNK):
        cols = pl.ds(j * _CHUNK, _CHUNK)
        xc = x_ref[cols, :]
        ht = jnp.maximum(
            jax.lax.dot_general(
                w1, xc, dimension_numbers=(((0,), (1,)), ((), ())),
                precision=jax.lax.Precision.DEFAULT,
                preferred_element_type=jnp.float32) + b1, 0.0)
        o_ref[:, cols] = _dot_t(w2, ht) + b2


def kernel(inputs, enc_w1, enc_b1, enc_w2, enc_b2,
           dec_w1, dec_b1, dec_w2, dec_b2, prior):
    del dec_w1, dec_b1, dec_w2, dec_b2, prior
    b, feat = inputs.shape
    hid = enc_w1.shape[1]
    code = enc_w2.shape[1]
    grid = (b // _BM,)
    out_t = pl.pallas_call(
        _encoder_body,
        grid=grid,
        in_specs=[
            pl.BlockSpec((_BM, feat), lambda i: (i, 0)),
            pl.BlockSpec((feat, hid), lambda i: (0, 0)),
            pl.BlockSpec((hid, 1), lambda i: (0, 0)),
            pl.BlockSpec((hid, code), lambda i: (0, 0)),
            pl.BlockSpec((code, 1), lambda i: (0, 0)),
        ],
        out_specs=pl.BlockSpec((code, _BM), lambda i: (0, i)),
        out_shape=jax.ShapeDtypeStruct((code, b), jnp.float32),
        compiler_params=pltpu.CompilerParams(
            dimension_semantics=("parallel",),
        ),
    )(inputs, enc_w1, enc_b1.reshape(hid, 1),
      enc_w2, enc_b2.reshape(code, 1))
    return out_t.T


# speedup vs baseline: 1.1193x; 1.1193x over previous
import jax
import jax.numpy as jnp
from jax.experimental import pallas as pl
from jax.experimental.pallas import tpu as pltpu

_BM = 2048
_CHUNK = 2048


def _dot_t(w, a):
    return jax.lax.dot_general(
        w, a, dimension_numbers=(((0,), (0,)), ((), ())),
        precision=jax.lax.Precision.DEFAULT,
        preferred_element_type=jnp.float32)


def _encoder_body(x_ref, w1_ref, b1_ref, w2_ref, b2_ref, o_ref):
    w1 = w1_ref[...]
    b1 = b1_ref[...]
    w2 = w2_ref[...]
    b2 = b2_ref[...]
    for j in range(_BM // _CHUNK):
        cols = pl.ds(j * _CHUNK, _CHUNK)
        xc = x_ref[cols, :]
        ht = jnp.maximum(
            jax.lax.dot_general(
                w1, xc, dimension_numbers=(((0,), (1,)), ((), ())),
                precision=jax.lax.Precision.DEFAULT,
                preferred_element_type=jnp.float32) + b1, 0.0)
        o_ref[:, cols] = _dot_t(w2, ht) + b2


def kernel(inputs, enc_w1, enc_b1, enc_w2, enc_b2,
           dec_w1, dec_b1, dec_w2, dec_b2, prior):
    del dec_w1, dec_b1, dec_w2, dec_b2, prior
    b, feat = inputs.shape
    hid = enc_w1.shape[1]
    code = enc_w2.shape[1]
    grid = (b // _BM,)
    out_t = pl.pallas_call(
        _encoder_body,
        grid=grid,
        in_specs=[
            pl.BlockSpec((_BM, feat), lambda i: (i, 0)),
            pl.BlockSpec((feat, hid), lambda i: (0, 0)),
            pl.BlockSpec((hid, 1), lambda i: (0, 0)),
            pl.BlockSpec((hid, code), lambda i: (0, 0)),
            pl.BlockSpec((code, 1), lambda i: (0, 0)),
        ],
        out_specs=pl.BlockSpec((code, _BM), lambda i: (0, i)),
        out_shape=jax.ShapeDtypeStruct((code, b), jnp.float32),
        compiler_params=pltpu.CompilerParams(
            dimension_semantics=("parallel",),
        ),
    )(inputs, enc_w1, enc_b1.reshape(hid, 1),
      enc_w2, enc_b2.reshape(code, 1))
    return out_t.T
